# Initial kernel scaffold; baseline (speedup 1.0000x reference)
import functools

import jax
import jax.numpy as jnp
from jax import lax
from jax.experimental import pallas as pl
from jax.experimental.pallas import tpu as pltpu
from jax.experimental.pallas import tpu_sc as plsc

CHUNK = 128
K = 4
STEP = CHUNK * K


@functools.partial(jax.jit, static_argnames=("n_rows", "hdim"))
def _emb_lookup(x2d, table, n_rows, hdim):
    info = plsc.get_sparse_core_info()
    nc, ns = info.num_cores, info.num_subcores
    nw = nc * ns
    rpw = n_rows // nw
    cpw = rpw // CHUNK
    steps = rpw // STEP

    @functools.partial(
        pl.kernel,
        mesh=plsc.VectorSubcoreMesh(core_axis_name="c", subcore_axis_name="s"),
        out_type=jax.ShapeDtypeStruct((n_rows, hdim), jnp.float32),
        scratch_types=[
            pltpu.VMEM((cpw, CHUNK), jnp.int32),
            pltpu.VMEM((STEP, hdim), jnp.float32),
            pltpu.SemaphoreType.DMA,
        ],
    )
    def body(x2_hbm, table_hbm, out_hbm, idx_v, rows_v, gsem):
        wid = lax.axis_index("s") * nc + lax.axis_index("c")
        cbase = wid * cpw
        obase = wid * rpw
        pltpu.sync_copy(x2_hbm.at[pl.ds(cbase, cpw)], idx_v)

        def step_fn(g, carry):
            cps = [
                pltpu.async_copy(
                    table_hbm.at[idx_v.at[g * K + j]],
                    rows_v.at[pl.ds(j * CHUNK, CHUNK)],
                    gsem,
                )
                for j in range(K)
            ]
            for cp in cps:
                cp.wait()
            pltpu.sync_copy(rows_v, out_hbm.at[pl.ds(obase + g * STEP, STEP)])
            return carry

        lax.fori_loop(0, steps, step_fn, 0)

    return body(x2d, table)


def kernel(x, table):
    b, h = x.shape
    n = b * h
    hdim = table.shape[1]
    x2d = x.astype(jnp.int32).reshape(n // CHUNK, CHUNK)
    out = _emb_lookup(x2d, table, n_rows=n, hdim=hdim)
    return out.reshape(b, h, hdim), None


# speedup vs baseline: 1.8328x; 1.8328x over previous
import functools

import jax
import jax.numpy as jnp
from jax import lax
from jax.experimental import pallas as pl
from jax.experimental.pallas import tpu as pltpu
from jax.experimental.pallas import tpu_sc as plsc

CHUNK = 128
K = 4
STEP = CHUNK * K


@functools.partial(jax.jit, static_argnames=("n_rows", "hdim"))
def _emb_lookup(x2d, table, n_rows, hdim):
    info = plsc.get_sparse_core_info()
    nc, ns = info.num_cores, info.num_subcores
    nw = nc * ns
    rpw = n_rows // nw
    cpw = rpw // CHUNK
    steps = rpw // STEP

    @functools.partial(
        pl.kernel,
        mesh=plsc.VectorSubcoreMesh(core_axis_name="c", subcore_axis_name="s"),
        out_type=jax.ShapeDtypeStruct((n_rows, hdim), jnp.float32),
        scratch_types=[
            pltpu.VMEM((cpw, CHUNK), jnp.int32),
            pltpu.VMEM((STEP, hdim), jnp.float32),
            pltpu.SemaphoreType.DMA,
        ],
        compiler_params=pltpu.CompilerParams(use_tc_tiling_on_sc=False),
    )
    def body(x2_hbm, table_hbm, out_hbm, idx_v, rows_v, gsem):
        wid = lax.axis_index("s") * nc + lax.axis_index("c")
        cbase = wid * cpw
        obase = wid * rpw
        pltpu.sync_copy(x2_hbm.at[pl.ds(cbase, cpw)], idx_v)

        def step_fn(g, carry):
            cps = [
                pltpu.async_copy(
                    table_hbm.at[idx_v.at[g * K + j]],
                    rows_v.at[pl.ds(j * CHUNK, CHUNK)],
                    gsem,
                )
                for j in range(K)
            ]
            for cp in cps:
                cp.wait()
            pltpu.sync_copy(rows_v, out_hbm.at[pl.ds(obase + g * STEP, STEP)])
            return carry

        lax.fori_loop(0, steps, step_fn, 0)

    return body(x2d, table)


def kernel(x, table):
    b, h = x.shape
    n = b * h
    hdim = table.shape[1]
    x2d = x.astype(jnp.int32).reshape(n // CHUNK, CHUNK)
    out = _emb_lookup(x2d, table, n_rows=n, hdim=hdim)
    return out.reshape(b, h, hdim), None


# speedup vs baseline: 1.8714x; 1.0211x over previous
import functools

import jax
import jax.numpy as jnp
from jax import lax
from jax.experimental import pallas as pl
from jax.experimental.pallas import tpu as pltpu
from jax.experimental.pallas import tpu_sc as plsc

CHUNK = 128
K = 4
STEP = CHUNK * K


@functools.partial(jax.jit, static_argnames=("n_rows", "hdim"))
def _emb_lookup(x2d, table, n_rows, hdim):
    info = plsc.get_sparse_core_info()
    nc, ns = info.num_cores, info.num_subcores
    nw = nc * ns
    rpw = n_rows // nw
    cpw = rpw // CHUNK
    steps = rpw // STEP

    @functools.partial(
        pl.kernel,
        mesh=plsc.VectorSubcoreMesh(core_axis_name="c", subcore_axis_name="s"),
        out_type=jax.ShapeDtypeStruct((n_rows, hdim), jnp.float32),
        scratch_types=[
            pltpu.VMEM((cpw, CHUNK), jnp.int32),
            pltpu.VMEM((STEP, hdim), jnp.float32),
            pltpu.VMEM((STEP, hdim), jnp.float32),
            pltpu.SemaphoreType.DMA,
            pltpu.SemaphoreType.DMA,
            pltpu.SemaphoreType.DMA,
            pltpu.SemaphoreType.DMA,
        ],
        compiler_params=pltpu.CompilerParams(use_tc_tiling_on_sc=False),
    )
    def body(x2_hbm, table_hbm, out_hbm, idx_v, rows0, rows1,
             gsem0, gsem1, ssem0, ssem1):
        wid = lax.axis_index("s") * nc + lax.axis_index("c")
        cbase = wid * cpw
        obase = wid * rpw
        pltpu.sync_copy(x2_hbm.at[pl.ds(cbase, cpw)], idx_v)

        def fire_gathers(g, buf, sem):
            for j in range(K):
                pltpu.async_copy(
                    table_hbm.at[idx_v.at[g * K + j]],
                    buf.at[pl.ds(j * CHUNK, CHUNK)],
                    sem,
                )

        def drain(sem, buf):
            pltpu.make_async_copy(table_hbm.at[pl.ds(0, STEP)], buf, sem).wait()

        def fire_store(g, buf, sem):
            pltpu.async_copy(buf, out_hbm.at[pl.ds(obase + g * STEP, STEP)], sem)

        fire_gathers(0, rows0, gsem0)

        def pair_fn(p, carry):
            g0 = 2 * p
            g1 = g0 + 1
            drain(gsem0, rows0)

            @pl.when(p > 0)
            def _():
                drain(ssem1, rows1)

            fire_gathers(g1, rows1, gsem1)
            fire_store(g0, rows0, ssem0)
            drain(gsem1, rows1)
            drain(ssem0, rows0)

            @pl.when(g1 + 1 < steps)
            def _():
                fire_gathers(g1 + 1, rows0, gsem0)

            fire_store(g1, rows1, ssem1)
            return carry

        lax.fori_loop(0, steps // 2, pair_fn, 0)
        drain(ssem1, rows1)

    return body(x2d, table)


def kernel(x, table):
    b, h = x.shape
    n = b * h
    hdim = table.shape[1]
    x2d = x.astype(jnp.int32).reshape(n // CHUNK, CHUNK)
    out = _emb_lookup(x2d, table, n_rows=n, hdim=hdim)
    return out.reshape(b, h, hdim), None
